# baseline (speedup 1.0000x reference)
import functools

import jax
import jax.numpy as jnp
from jax.experimental import pallas as pl
from jax.experimental.pallas import tpu as pltpu

_EPS = 1e-5


def _stats_kernel(x_ref, gram_ref, *, cin):
    x = x_ref[...]
    ch = [x[:, i, :] for i in range(cin)]
    parts = [ch[i] for i in range(cin)]
    parts += [ch[i] * ch[k] for i in range(cin) for k in range(i, cin)]
    lane = jax.lax.broadcasted_iota(jnp.int32, gram_ref.shape[-2:], 1)
    z = jnp.zeros(gram_ref.shape[-2:], jnp.float32)
    for r, t in enumerate(parts):
        z = jnp.where(lane == r, jnp.sum(t), z)
    gram_ref[...] = z


def _conv_bn_relu_kernel(x_ref, wblk_ref, shift_ref, o_ref):
    y = jnp.dot(wblk_ref[...], x_ref[...], preferred_element_type=jnp.float32)
    o_ref[...] = jnp.maximum(y + shift_ref[...], 0.0)


@jax.jit
def kernel(x_nchw, conv_w, conv_b, bn_gamma, bn_beta):
    n, cin, h, w = x_nchw.shape
    cout = conv_w.shape[0]
    ho, wo = (h + 1) // 2, (w + 1) // 2
    p = ho * wo
    del conv_b
    w2 = conv_w.reshape(cout, cin).astype(jnp.float32)

    x2 = x_nchw[:, :, ::2, ::2].reshape(n, cin, p)
    x2r = x2.reshape(n * cin, p)

    nb = next(d for d in (8, 4, 2, 1) if n % (2 * d) == 0)
    g1 = n // nb
    stat_cols = 16

    gram = pl.pallas_call(
        functools.partial(_stats_kernel, cin=cin),
        out_shape=jax.ShapeDtypeStruct((g1, 1, stat_cols), jnp.float32),
        grid=(g1,),
        in_specs=[pl.BlockSpec((nb, cin, p), lambda i: (i, 0, 0))],
        out_specs=pl.BlockSpec((None, 1, stat_cols), lambda i: (i, 0, 0)),
        compiler_params=pltpu.CompilerParams(
            dimension_semantics=("parallel",)),
        name="x_moments",
    )(x2)

    g = jnp.sum(gram, axis=(0, 1))
    s = g[:cin]
    iu = jnp.triu_indices(cin)
    m_up = jnp.zeros((cin, cin), jnp.float32).at[iu].set(
        g[cin:cin + (cin * (cin + 1)) // 2])
    m_full = m_up + m_up.T - jnp.diag(jnp.diag(m_up))
    inv_count = 1.0 / float(n * p)
    mean_y = (w2 @ s) * inv_count
    ey2 = jnp.einsum("oc,cd,od->o", w2, m_full, w2) * inv_count
    var = jnp.maximum(ey2 - mean_y * mean_y, 0.0)
    scale = bn_gamma * jax.lax.rsqrt(var + _EPS)
    shift = bn_beta - mean_y * scale
    wf = scale[:, None] * w2

    nb2 = nb
    g2 = n // nb2
    wblk = jnp.kron(jnp.eye(nb2, dtype=jnp.float32), wf)
    shift_blk = jnp.tile(shift[:, None], (nb2, 1))

    out_flat = pl.pallas_call(
        _conv_bn_relu_kernel,
        out_shape=jax.ShapeDtypeStruct((n * cout, p), jnp.float32),
        grid=(g2,),
        in_specs=[
            pl.BlockSpec((nb2 * cin, p), lambda i: (i, 0)),
            pl.BlockSpec((nb2 * cout, nb2 * cin), lambda i: (0, 0)),
            pl.BlockSpec((nb2 * cout, 1), lambda i: (0, 0)),
        ],
        out_specs=pl.BlockSpec((nb2 * cout, p), lambda i: (i, 0)),
        compiler_params=pltpu.CompilerParams(
            dimension_semantics=("parallel",)),
        name="folded_conv_bn_relu",
    )(x2r, wblk, shift_blk)

    return out_flat.reshape(n, cout, ho, wo)


# speedup vs baseline: 1.5003x; 1.5003x over previous
import functools

import jax
import jax.numpy as jnp
from jax.experimental import pallas as pl
from jax.experimental.pallas import tpu as pltpu

_EPS = 1e-5


def _compact_stats_kernel(x_ref, selw_ref, x2_ref, gram_ref, *, nb, cin, ho,
                          wo):
    acc = [None] * 9
    for b in range(nb):
        vb = x_ref[b].astype(jnp.bfloat16)
        xc = jnp.dot(vb, selw_ref[...],
                     preferred_element_type=jnp.float32)
        x2_ref[b] = xc.reshape(cin, ho, wo).astype(jnp.bfloat16)
        ch = [xc[i * ho:(i + 1) * ho] for i in range(cin)]
        parts = [ch[i] for i in range(cin)]
        parts += [ch[i] * ch[k] for i in range(cin) for k in range(i, cin)]
        for r, t in enumerate(parts):
            s = jnp.sum(t)
            acc[r] = s if acc[r] is None else acc[r] + s

    lane = jax.lax.broadcasted_iota(jnp.int32, gram_ref.shape[-2:], 1)
    z = jnp.zeros(gram_ref.shape[-2:], jnp.float32)
    for r, s in enumerate(acc):
        z = jnp.where(lane == r, s, z)
    gram_ref[...] = z


def _conv_bn_relu_kernel(x_ref, wblk_ref, shift_ref, o_ref):
    y = jnp.dot(wblk_ref[...], x_ref[...], preferred_element_type=jnp.float32)
    o_ref[...] = jnp.maximum(y + shift_ref[...], 0.0)


@jax.jit
def kernel(x_nchw, conv_w, conv_b, bn_gamma, bn_beta):
    n, cin, h, w = x_nchw.shape
    cout = conv_w.shape[0]
    ho, wo = (h + 1) // 2, (w + 1) // 2
    p = ho * wo
    del conv_b
    w2 = conv_w.reshape(cout, cin).astype(jnp.float32)

    xh = x_nchw[:, :, ::2, :].reshape(n, cin * ho, w)

    nb = next(d for d in (8, 4, 2, 1) if n % (2 * d) == 0)
    g1 = n // nb
    stat_cols = 16

    selw = (jax.lax.broadcasted_iota(jnp.int32, (w, wo), 0) ==
            2 * jax.lax.broadcasted_iota(jnp.int32, (w, wo), 1)
            ).astype(jnp.bfloat16)

    x2c, gram = pl.pallas_call(
        functools.partial(_compact_stats_kernel, nb=nb, cin=cin, ho=ho,
                          wo=wo),
        out_shape=(jax.ShapeDtypeStruct((n, cin, ho, wo), jnp.bfloat16),
                   jax.ShapeDtypeStruct((g1, 1, stat_cols), jnp.float32)),
        grid=(g1,),
        in_specs=[pl.BlockSpec((nb, cin * ho, w), lambda i: (i, 0, 0)),
                  pl.BlockSpec((w, wo), lambda i: (0, 0))],
        out_specs=(
            pl.BlockSpec((nb, cin, ho, wo), lambda i: (i, 0, 0, 0)),
            pl.BlockSpec((None, 1, stat_cols), lambda i: (i, 0, 0)),
        ),
        compiler_params=pltpu.CompilerParams(
            dimension_semantics=("parallel",)),
        name="compact_stats",
    )(xh, selw)

    g = jnp.sum(gram, axis=(0, 1))
    s = g[:cin]
    iu = jnp.triu_indices(cin)
    m_up = jnp.zeros((cin, cin), jnp.float32).at[iu].set(
        g[cin:cin + (cin * (cin + 1)) // 2])
    m_full = m_up + m_up.T - jnp.diag(jnp.diag(m_up))
    inv_count = 1.0 / float(n * p)
    mean_y = (w2 @ s) * inv_count
    ey2 = jnp.einsum("oc,cd,od->o", w2, m_full, w2) * inv_count
    var = jnp.maximum(ey2 - mean_y * mean_y, 0.0)
    scale = bn_gamma * jax.lax.rsqrt(var + _EPS)
    shift = bn_beta - mean_y * scale
    wf = scale[:, None] * w2

    nb2 = nb
    g2 = n // nb2
    wblk = jnp.kron(jnp.eye(nb2, dtype=jnp.float32), wf).astype(jnp.bfloat16)
    shift_blk = jnp.tile(shift[:, None], (nb2, 1))
    x2r = x2c.reshape(n * cin, p)

    out_flat = pl.pallas_call(
        _conv_bn_relu_kernel,
        out_shape=jax.ShapeDtypeStruct((n * cout, p), jnp.float32),
        grid=(g2,),
        in_specs=[
            pl.BlockSpec((nb2 * cin, p), lambda i: (i, 0)),
            pl.BlockSpec((nb2 * cout, nb2 * cin), lambda i: (0, 0)),
            pl.BlockSpec((nb2 * cout, 1), lambda i: (0, 0)),
        ],
        out_specs=pl.BlockSpec((nb2 * cout, p), lambda i: (i, 0)),
        compiler_params=pltpu.CompilerParams(
            dimension_semantics=("parallel",)),
        name="folded_conv_bn_relu",
    )(x2r, wblk, shift_blk)

    return out_flat.reshape(n, cout, ho, wo)
